# Initial kernel scaffold; baseline (speedup 1.0000x reference)
import jax
import jax.numpy as jnp
from jax.experimental import pallas as pl


def kernel(inputs, W1, b1, W2, b2, W3, b3, emb, D1, db1, D2, db2, D3, db3):
    raise NotImplementedError("write your pallas kernel here")


# speedup vs baseline: 1.0392x; 1.0392x over previous
import functools

import jax
import jax.numpy as jnp
from jax import lax
from jax.experimental import pallas as pl
from jax.experimental.pallas import tpu as pltpu
from jax.experimental.pallas import tpu_sc as plsc

COM_COST = 0.25


def _linear_kernel(x_ref, w_ref, b_ref, o_ref, *, act):
    acc = jnp.dot(x_ref[...].astype(jnp.bfloat16), w_ref[...].astype(jnp.bfloat16),
                  preferred_element_type=jnp.float32)
    acc = acc + b_ref[...][None, :]
    if act:
        acc = jnp.tanh(acc)
    o_ref[...] = acc


def _linear(x, w, b, act=True, rb=256):
    m, k = x.shape
    k2, n = w.shape
    assert k == k2
    return pl.pallas_call(
        functools.partial(_linear_kernel, act=act),
        grid=(m // rb,),
        in_specs=[
            pl.BlockSpec((rb, k), lambda i: (i, 0)),
            pl.BlockSpec((k, n), lambda i: (0, 0)),
            pl.BlockSpec((n,), lambda i: (0,)),
        ],
        out_specs=pl.BlockSpec((rb, n), lambda i: (i, 0)),
        out_shape=jax.ShapeDtypeStruct((m, n), jnp.float32),
        compiler_params=pltpu.CompilerParams(
            dimension_semantics=("parallel",)),
    )(x, w, b)


def _argmin_kernel(z_ref, emb_ref, esq_ref, idx_ref):
    z = z_ref[...]
    zsq = jnp.sum(z * z, axis=1, keepdims=True)
    mm = lax.dot_general(
        z.astype(jnp.bfloat16), emb_ref[...].astype(jnp.bfloat16),
        (((1,), (1,)), ((), ())), preferred_element_type=jnp.float32)
    d = (zsq + esq_ref[...]) - 2.0 * mm
    dmin = jnp.min(d, axis=1, keepdims=True)
    iota = lax.broadcasted_iota(jnp.int32, d.shape, 1)
    idx = jnp.min(jnp.where(d == dmin, iota, jnp.int32(2**30)), axis=1)
    idx_ref[...] = idx[None, None, :]


def _vq_argmin(z, emb, esq, rb=256):
    m, c = z.shape
    k = emb.shape[0]
    out = pl.pallas_call(
        _argmin_kernel,
        grid=(m // rb,),
        in_specs=[
            pl.BlockSpec((rb, c), lambda i: (i, 0)),
            pl.BlockSpec((k, c), lambda i: (0, 0)),
            pl.BlockSpec((1, k), lambda i: (0, 0)),
        ],
        out_specs=pl.BlockSpec((1, 1, rb), lambda i: (i, 0, 0)),
        out_shape=jax.ShapeDtypeStruct((m // rb, 1, rb), jnp.int32),
        compiler_params=pltpu.CompilerParams(
            dimension_semantics=("parallel",)),
    )(z, emb, esq)
    return out.reshape(m)


def _esq_kernel(emb_ref, o_ref):
    e = emb_ref[...]
    o_ref[...] = jnp.sum(e * e, axis=1)[None, :]


def _emb_sq_norms(emb):
    k, c = emb.shape
    return pl.pallas_call(
        _esq_kernel,
        in_specs=[pl.BlockSpec((k, c), lambda: (0, 0))],
        out_specs=pl.BlockSpec((1, k), lambda: (0, 0)),
        out_shape=jax.ShapeDtypeStruct((1, k), jnp.float32),
    )(emb)


def _sc_gather(table, idx):
    v, d = table.shape
    b = idx.shape[0]
    info = plsc.get_sparse_core_info()
    nw = info.num_cores * info.num_subcores
    b_per_w = b // nw
    mesh = plsc.VectorSubcoreMesh(core_axis_name="c", subcore_axis_name="s")

    @functools.partial(
        pl.kernel, mesh=mesh,
        out_type=jax.ShapeDtypeStruct((b, d), jnp.float32),
        scratch_types=[
            pltpu.VMEM((b_per_w,), jnp.int32),
            pltpu.VMEM((b_per_w, d), jnp.float32),
            pltpu.SemaphoreType.DMA,
        ],
    )
    def k(table_hbm, idx_hbm, out_hbm, idx_v, rows_v, sem):
        wid = lax.axis_index("s") * info.num_cores + lax.axis_index("c")
        base = wid * b_per_w
        pltpu.sync_copy(idx_hbm.at[pl.ds(base, b_per_w)], idx_v)
        pltpu.async_copy(table_hbm.at[idx_v], rows_v, sem).wait()
        pltpu.sync_copy(rows_v, out_hbm.at[pl.ds(base, b_per_w)])

    return k(table, idx)


def _qst_kernel(z_ref, q_ref, qst_ref, lp_ref):
    z = z_ref[...]
    q = q_ref[...]
    diff = q - z
    qst_ref[...] = z + diff
    lp_ref[...] = jnp.sum(diff * diff).reshape(1, 1, 1)


def _qst_loss(z, q, rb=256):
    m, c = z.shape
    qst, lp = pl.pallas_call(
        _qst_kernel,
        grid=(m // rb,),
        in_specs=[
            pl.BlockSpec((rb, c), lambda i: (i, 0)),
            pl.BlockSpec((rb, c), lambda i: (i, 0)),
        ],
        out_specs=[
            pl.BlockSpec((rb, c), lambda i: (i, 0)),
            pl.BlockSpec((1, 1, 1), lambda i: (i, 0, 0)),
        ],
        out_shape=[
            jax.ShapeDtypeStruct((m, c), jnp.float32),
            jax.ShapeDtypeStruct((m // rb, 1, 1), jnp.float32),
        ],
        compiler_params=pltpu.CompilerParams(
            dimension_semantics=("parallel",)),
    )(z, q)
    mean_sq = jnp.sum(lp) / (m * c)
    return qst, mean_sq


def kernel(inputs, W1, b1, W2, b2, W3, b3, emb, D1, db1, D2, db2, D3, db3):
    z = _linear(inputs, W1, b1)
    z = _linear(z, W2, b2)
    z = _linear(z, W3, b3)
    esq = _emb_sq_norms(emb)
    encoding_indices = _vq_argmin(z, emb, esq)
    quantized = _sc_gather(emb, encoding_indices)
    quantized_st, e_latent_loss = _qst_loss(z, quantized)
    loss = e_latent_loss + COM_COST * e_latent_loss
    h = _linear(quantized_st, D1, db1)
    h = _linear(h, D2, db2)
    x_recon = _linear(h, D3, db3, act=False)
    return (loss, x_recon, quantized_st)
